# Initial kernel scaffold; baseline (speedup 1.0000x reference)
import jax
import jax.numpy as jnp
from jax.experimental import pallas as pl


def kernel(input_, weight):
    raise NotImplementedError("write your pallas kernel here")


# speedup vs baseline: 1.1120x; 1.1120x over previous
import functools

import jax
import jax.numpy as jnp
from jax import lax
from jax.experimental import pallas as pl
from jax.experimental.pallas import tpu as pltpu
from jax.experimental.pallas import tpu_sc as plsc

NUM_EMBEDDINGS = 1000000
DIM = 32
TOT = 16384 * 100
NC = 2
NS = 16
NW = NC * NS
PER_W = TOT // NW
CHUNK = 128
NCHUNK = PER_W // CHUNK
NBUF = 8
DEPTH = 4
NG = NCHUNK // NBUF

_mesh = plsc.VectorSubcoreMesh(core_axis_name="c", subcore_axis_name="s")


@functools.partial(
    pl.kernel,
    mesh=_mesh,
    out_type=jax.ShapeDtypeStruct((TOT, DIM), jnp.float32),
    compiler_params=pltpu.CompilerParams(use_tc_tiling_on_sc=False),
    scratch_types=[
        pltpu.VMEM((NCHUNK, CHUNK), jnp.int32),
        pltpu.VMEM((NBUF, CHUNK, DIM), jnp.float32),
        pltpu.SemaphoreType.DMA,
        pltpu.SemaphoreType.DMA,
    ],
)
def _emb_lookup(idx_hbm, table_hbm, out_hbm, idx_v, rows_v, gsem, osem):
    wid = lax.axis_index("s") * NC + lax.axis_index("c")
    base = wid * PER_W

    pltpu.sync_copy(idx_hbm.at[wid], idx_v)

    def start_gather(j, slot):
        pltpu.async_copy(table_hbm.at[idx_v.at[j]], rows_v.at[slot], gsem)

    def wait_gather(j, slot):
        pltpu.make_async_copy(
            table_hbm.at[idx_v.at[j]], rows_v.at[slot], gsem
        ).wait()

    def start_store(j, slot):
        pltpu.async_copy(
            rows_v.at[slot], out_hbm.at[pl.ds(base + j * CHUNK, CHUNK)], osem
        )

    def wait_store(j, slot):
        pltpu.make_async_copy(
            rows_v.at[slot], out_hbm.at[pl.ds(base + j * CHUNK, CHUNK)], osem
        ).wait()

    for b in range(NBUF):
        start_gather(b, b)
        if b >= DEPTH:
            wait_gather(b - DEPTH, b - DEPTH)
            start_store(b - DEPTH, b - DEPTH)

    def round_body(g, carry):
        for b in range(NBUF):
            j = g * NBUF + b
            wait_store(j - NBUF, b)
            start_gather(j, b)
            jd = j - DEPTH
            sd = (b - DEPTH) % NBUF
            wait_gather(jd, sd)
            start_store(jd, sd)
        return carry

    lax.fori_loop(1, NG, round_body, 0)

    for b in range(NBUF):
        j = NG * NBUF + b
        wait_store(j - NBUF, b)
        if b < DEPTH:
            jd = j - DEPTH
            sd = (b - DEPTH) % NBUF
            wait_gather(jd, sd)
            start_store(jd, sd)


def kernel(input_, weight):
    idx = input_.astype(jnp.int32).reshape(NW, NCHUNK, CHUNK)
    out = _emb_lookup(idx, weight)
    return out.reshape(input_.shape[0], input_.shape[1], DIM)


# speedup vs baseline: 1.1125x; 1.0004x over previous
import functools

import jax
import jax.numpy as jnp
from jax import lax
from jax.experimental import pallas as pl
from jax.experimental.pallas import tpu as pltpu
from jax.experimental.pallas import tpu_sc as plsc

NUM_EMBEDDINGS = 1000000
DIM = 32
TOT = 16384 * 100
NC = 2
NS = 16
NW = NC * NS
PER_W = TOT // NW
CHUNK = 512
NCHUNK = PER_W // CHUNK
NBUF = 4
DEPTH = 2
NG = NCHUNK // NBUF

_mesh = plsc.VectorSubcoreMesh(core_axis_name="c", subcore_axis_name="s")


@functools.partial(
    pl.kernel,
    mesh=_mesh,
    out_type=jax.ShapeDtypeStruct((TOT, DIM), jnp.float32),
    compiler_params=pltpu.CompilerParams(use_tc_tiling_on_sc=False),
    scratch_types=[
        pltpu.VMEM((NCHUNK, CHUNK), jnp.int32),
        pltpu.VMEM((NBUF, CHUNK, DIM), jnp.float32),
        pltpu.SemaphoreType.DMA,
        pltpu.SemaphoreType.DMA,
    ],
)
def _emb_lookup(idx_hbm, table_hbm, out_hbm, idx_v, rows_v, gsem, osem):
    wid = lax.axis_index("s") * NC + lax.axis_index("c")
    base = wid * PER_W

    pltpu.sync_copy(idx_hbm.at[wid], idx_v)

    def start_gather(j, slot):
        pltpu.async_copy(table_hbm.at[idx_v.at[j]], rows_v.at[slot], gsem)

    def wait_gather(j, slot):
        pltpu.make_async_copy(
            table_hbm.at[idx_v.at[j]], rows_v.at[slot], gsem
        ).wait()

    def start_store(j, slot):
        pltpu.async_copy(
            rows_v.at[slot], out_hbm.at[pl.ds(base + j * CHUNK, CHUNK)], osem
        )

    def wait_store(j, slot):
        pltpu.make_async_copy(
            rows_v.at[slot], out_hbm.at[pl.ds(base + j * CHUNK, CHUNK)], osem
        ).wait()

    for b in range(NBUF):
        start_gather(b, b)
        if b >= DEPTH:
            wait_gather(b - DEPTH, b - DEPTH)
            start_store(b - DEPTH, b - DEPTH)

    def round_body(g, carry):
        for b in range(NBUF):
            j = g * NBUF + b
            wait_store(j - NBUF, b)
            start_gather(j, b)
            jd = j - DEPTH
            sd = (b - DEPTH) % NBUF
            wait_gather(jd, sd)
            start_store(jd, sd)
        return carry

    lax.fori_loop(1, NG, round_body, 0)

    for b in range(NBUF):
        j = NG * NBUF + b
        wait_store(j - NBUF, b)
        if b < DEPTH:
            jd = j - DEPTH
            sd = (b - DEPTH) % NBUF
            wait_gather(jd, sd)
            start_store(jd, sd)


def kernel(input_, weight):
    idx = input_.astype(jnp.int32).reshape(NW, NCHUNK, CHUNK)
    out = _emb_lookup(idx, weight)
    return out.reshape(input_.shape[0], input_.shape[1], DIM)
